# baseline (speedup 1.0000x reference)
import functools

import jax
import jax.numpy as jnp
from jax import lax
from jax.experimental import pallas as pl
from jax.experimental.pallas import tpu as pltpu
from jax.experimental.pallas import tpu_sc as plsc

B = 4096
NC = 2
NS = 16
NW = NC * NS
BPW = B // NW
L = 16

DG = 3
DP = 69
DT = 3
DB = 10


def _body(idx_hbm, betas_hbm, go_hbm, bp_hbm, tr_hbm,
          out_b, out_go, out_bp, out_tr,
          idx_v, ego, ebp, bet_v, b_rows, go_rows, bp_rows, tr_rows,
          sem, osem):
    wid = lax.axis_index("s") * NC + lax.axis_index("c")
    base = wid * BPW

    pltpu.sync_copy(idx_hbm.at[pl.ds(base, BPW)], idx_v)
    pltpu.sync_copy(betas_hbm, bet_v)

    iota = lax.iota(jnp.int32, L)

    def row(b, _):
        ivec = plsc.load_gather(idx_v, [jnp.full((L,), b, jnp.int32)])
        plsc.store_scatter(ego, [DG * b + iota], DG * ivec + iota)
        for k in range(5):
            plsc.store_scatter(ebp, [DP * b + 16 * k + iota],
                               DP * ivec + 16 * k + iota)
        return _

    lax.fori_loop(0, BPW, row, None)

    cps = [
        pltpu.async_copy(go_hbm.at[ego.at[pl.ds(0, BPW * DG)]], go_rows, sem),
        pltpu.async_copy(tr_hbm.at[ego.at[pl.ds(0, BPW * DG)]], tr_rows, sem),
        pltpu.async_copy(bp_hbm.at[ebp.at[pl.ds(0, BPW * DP)]], bp_rows, sem),
    ]

    for m in range(5):
        lane = iota + 16 * m
        sel = jnp.where(lane >= 70, lane - 70,
                        jnp.where(lane >= 60, lane - 60,
                                  jnp.where(lane >= 50, lane - 50,
                                            jnp.where(lane >= 40, lane - 40,
                                                      jnp.where(lane >= 30, lane - 30,
                                                                jnp.where(lane >= 20, lane - 20,
                                                                          jnp.where(lane >= 10, lane - 10, lane)))))))
        vm = plsc.load_gather(bet_v, [sel])
        for r in range(BPW * DB // 80):
            b_rows[pl.ds(80 * r + 16 * m, L)] = vm

    for cp in cps:
        cp.wait()

    ocps = [
        pltpu.async_copy(go_rows, out_go.at[pl.ds(base * DG, BPW * DG)], osem),
        pltpu.async_copy(tr_rows, out_tr.at[pl.ds(base * DT, BPW * DT)], osem),
        pltpu.async_copy(bp_rows, out_bp.at[pl.ds(base * DP, BPW * DP)], osem),
        pltpu.async_copy(b_rows, out_b.at[pl.ds(base * DB, BPW * DB)], osem),
    ]
    for cp in ocps:
        cp.wait()


def kernel(idx, betas, global_orient, body_pose, transl):
    idx = idx.astype(jnp.int32)
    go_f = global_orient.reshape(-1)
    bp_f = body_pose.reshape(-1)
    tr_f = transl.reshape(-1)
    bet_f = betas.reshape(-1)
    mesh = plsc.VectorSubcoreMesh(core_axis_name="c", subcore_axis_name="s")
    run = functools.partial(
        pl.kernel,
        mesh=mesh,
        compiler_params=pltpu.CompilerParams(use_tc_tiling_on_sc=False,
                                             needs_layout_passes=False),
        out_type=[
            jax.ShapeDtypeStruct((B * DB,), jnp.float32),
            jax.ShapeDtypeStruct((B * DG,), jnp.float32),
            jax.ShapeDtypeStruct((B * DP,), jnp.float32),
            jax.ShapeDtypeStruct((B * DT,), jnp.float32),
        ],
        scratch_types=[
            pltpu.VMEM((BPW,), jnp.int32),
            pltpu.VMEM((BPW * DG + 16,), jnp.int32),
            pltpu.VMEM((BPW * DP + 16,), jnp.int32),
            pltpu.VMEM((DB,), jnp.float32),
            pltpu.VMEM((BPW * DB,), jnp.float32),
            pltpu.VMEM((BPW * DG,), jnp.float32),
            pltpu.VMEM((BPW * DP,), jnp.float32),
            pltpu.VMEM((BPW * DT,), jnp.float32),
            pltpu.SemaphoreType.DMA,
            pltpu.SemaphoreType.DMA,
        ],
    )(_body)
    ob, ogo, obp, otr = run(idx, bet_f, go_f, bp_f, tr_f)
    return (ob.reshape(B, DB), ogo.reshape(B, DG),
            obp.reshape(B, DP), otr.reshape(B, DT))
